# initial kernel scaffold (unmeasured)
import jax
import jax.numpy as jnp
from jax import lax
from jax.experimental import pallas as pl
from jax.experimental.pallas import tpu as pltpu


def kernel(Q, K, V):
    b, s, h, d = Q.shape
    hd = h * d
    rows = b * s
    scale = d ** -0.5

    def body(q_ref, k_ref, v_ref, o_ref, kv_ref, send_sem, recv_sem, exit_sem):
        my_x = lax.axis_index("x")
        my_y = lax.axis_index("y")
        my_z = lax.axis_index("z")
        nbr = (1 - my_x, my_y, my_z)

        kv_ref[0, 0] = k_ref[:, :].astype(jnp.bfloat16)
        kv_ref[0, 1] = v_ref[:, :].astype(jnp.bfloat16)

        barrier_sem = pltpu.get_barrier_semaphore()
        pl.semaphore_signal(
            barrier_sem, inc=1, device_id=nbr,
            device_id_type=pl.DeviceIdType.MESH,
        )
        pl.semaphore_wait(barrier_sem, 1)

        rdma = pltpu.make_async_remote_copy(
            src_ref=kv_ref.at[0],
            dst_ref=kv_ref.at[1],
            send_sem=send_sem,
            recv_sem=recv_sem,
            device_id=nbr,
            device_id_type=pl.DeviceIdType.MESH,
        )
        rdma.start()
        rdma.wait()

        qall = (q_ref[:, :] * scale).astype(jnp.bfloat16)
        k_loc = kv_ref[0, 0]
        v_loc = kv_ref[0, 1]
        k_rem = kv_ref[1, 0]
        v_rem = kv_ref[1, 1]

        dn_nt = (((1,), (1,)), ((), ()))
        dn_nn = (((1,), (0,)), ((), ()))

        for bi in range(b):
            r0 = bi * s
            head_outs = []
            for hi in range(h):
                c0 = hi * d
                q_bh = lax.slice(qall, (r0, c0), (r0 + s, c0 + d))
                k0 = lax.slice(k_loc, (r0, c0), (r0 + s, c0 + d))
                k1 = lax.slice(k_rem, (r0, c0), (r0 + s, c0 + d))
                v0 = lax.slice(v_loc, (r0, c0), (r0 + s, c0 + d))
                v1 = lax.slice(v_rem, (r0, c0), (r0 + s, c0 + d))
                s0 = lax.dot_general(q_bh, k0, dn_nt,
                                     preferred_element_type=jnp.float32)
                s1 = lax.dot_general(q_bh, k1, dn_nt,
                                     preferred_element_type=jnp.float32)
                m = jnp.maximum(jnp.max(s0, axis=1, keepdims=True),
                                jnp.max(s1, axis=1, keepdims=True))
                p0 = jnp.exp(s0 - m)
                p1 = jnp.exp(s1 - m)
                l = (jnp.sum(p0, axis=1, keepdims=True)
                     + jnp.sum(p1, axis=1, keepdims=True))
                o = (lax.dot_general(p0.astype(jnp.bfloat16), v0, dn_nn,
                                     preferred_element_type=jnp.float32)
                     + lax.dot_general(p1.astype(jnp.bfloat16), v1, dn_nn,
                                       preferred_element_type=jnp.float32))
                head_outs.append(o / l)
            o_ref[r0:r0 + s, :] = jnp.concatenate(head_outs, axis=1)

        pl.semaphore_signal(
            exit_sem, inc=1, device_id=nbr,
            device_id_type=pl.DeviceIdType.MESH,
        )
        pl.semaphore_wait(exit_sem, 1)

    out = pl.pallas_call(
        body,
        out_shape=jax.ShapeDtypeStruct((rows, hd), jnp.float32),
        in_specs=[pl.BlockSpec(memory_space=pltpu.VMEM)] * 3,
        out_specs=pl.BlockSpec(memory_space=pltpu.VMEM),
        scratch_shapes=[
            pltpu.VMEM((2, 2, rows, hd), jnp.bfloat16),
            pltpu.SemaphoreType.DMA,
            pltpu.SemaphoreType.DMA,
            pltpu.SemaphoreType.REGULAR,
        ],
        compiler_params=pltpu.CompilerParams(collective_id=0),
    )(Q.reshape(rows, hd), K.reshape(rows, hd), V.reshape(rows, hd))
    return out.reshape(b, s, h, d)


# baseline (device time: 60679 ns/iter reference)
import jax
import jax.numpy as jnp
from jax import lax
from jax.experimental import pallas as pl
from jax.experimental.pallas import tpu as pltpu


def kernel(Q, K, V):
    b, s, h, d = Q.shape
    hd = h * d
    rows = b * s
    scale = d ** -0.5

    def body(q_ref, k_ref, v_ref, o_ref, kv_ref, send_sem, recv_sem, exit_sem):
        my_x = lax.axis_index("x")
        my_y = lax.axis_index("y")
        my_z = lax.axis_index("z")
        nbr = (1 - my_x, my_y, my_z)

        kv_ref[0, 0] = k_ref[:, :].astype(jnp.bfloat16)
        kv_ref[0, 1] = v_ref[:, :].astype(jnp.bfloat16)

        barrier_sem = pltpu.get_barrier_semaphore()
        pl.semaphore_signal(
            barrier_sem, inc=1, device_id=nbr,
            device_id_type=pl.DeviceIdType.MESH,
        )
        pl.semaphore_wait(barrier_sem, 1)

        rdma = pltpu.make_async_remote_copy(
            src_ref=kv_ref.at[0],
            dst_ref=kv_ref.at[1],
            send_sem=send_sem,
            recv_sem=recv_sem,
            device_id=nbr,
            device_id_type=pl.DeviceIdType.MESH,
        )
        rdma.start()
        rdma.wait()

        qall = (q_ref[:, :] * scale).astype(jnp.bfloat16)
        k_loc = kv_ref[0, 0]
        v_loc = kv_ref[0, 1]
        k_rem = kv_ref[1, 0]
        v_rem = kv_ref[1, 1]

        dn_nt = (((1,), (1,)), ((), ()))
        dn_nn = (((1,), (0,)), ((), ()))

        for bi in range(b):
            r0 = bi * s
            head_outs = []
            for hi in range(h):
                c0 = hi * d
                q_bh = lax.slice(qall, (r0, c0), (r0 + s, c0 + d))
                k0 = lax.slice(k_loc, (r0, c0), (r0 + s, c0 + d))
                k1 = lax.slice(k_rem, (r0, c0), (r0 + s, c0 + d))
                v0 = lax.slice(v_loc, (r0, c0), (r0 + s, c0 + d))
                v1 = lax.slice(v_rem, (r0, c0), (r0 + s, c0 + d))
                s0 = lax.dot_general(q_bh, k0, dn_nt,
                                     preferred_element_type=jnp.float32)
                s1 = lax.dot_general(q_bh, k1, dn_nt,
                                     preferred_element_type=jnp.float32)
                m = jnp.maximum(jnp.max(s0, axis=1, keepdims=True),
                                jnp.max(s1, axis=1, keepdims=True))
                p0 = jnp.exp(s0 - m)
                p1 = jnp.exp(s1 - m)
                l = (jnp.sum(p0, axis=1, keepdims=True)
                     + jnp.sum(p1, axis=1, keepdims=True))
                o = (lax.dot_general(p0.astype(jnp.bfloat16), v0, dn_nn,
                                     preferred_element_type=jnp.float32)
                     + lax.dot_general(p1.astype(jnp.bfloat16), v1, dn_nn,
                                       preferred_element_type=jnp.float32))
                head_outs.append(o / l)
            o_ref[r0:r0 + s, :] = jnp.concatenate(head_outs, axis=1)

        pl.semaphore_signal(
            exit_sem, inc=1, device_id=nbr,
            device_id_type=pl.DeviceIdType.MESH,
        )
        pl.semaphore_wait(exit_sem, 1)

    out = pl.pallas_call(
        body,
        out_shape=jax.ShapeDtypeStruct((rows, hd), jnp.float32),
        in_specs=[pl.BlockSpec(memory_space=pltpu.VMEM)] * 3,
        out_specs=pl.BlockSpec(memory_space=pltpu.VMEM),
        scratch_shapes=[
            pltpu.VMEM((2, 2, rows, hd), jnp.bfloat16),
            pltpu.SemaphoreType.DMA,
            pltpu.SemaphoreType.DMA,
            pltpu.SemaphoreType.REGULAR,
        ],
        compiler_params=pltpu.CompilerParams(
            collective_id=0, vmem_limit_bytes=100 * 1024 * 1024
        ),
    )(Q.reshape(rows, hd), K.reshape(rows, hd), V.reshape(rows, hd))
    return out.reshape(b, s, h, d)


# device time: 34705 ns/iter; 1.7484x vs baseline; 1.7484x over previous
import os

import jax
import jax.numpy as jnp
from jax import lax
from jax.experimental import pallas as pl
from jax.experimental.pallas import tpu as pltpu

_KMODE = os.environ.get("KMODE", "full")


def kernel(Q, K, V):
    b, s, h, d = Q.shape
    scale = d ** -0.5
    comm = _KMODE != "nocomm"

    Qt = jnp.transpose(Q, (0, 2, 3, 1))
    Kt = jnp.transpose(K, (0, 2, 3, 1))
    Vt = jnp.transpose(V, (0, 2, 3, 1))

    def body(q_ref, k_ref, v_ref, o_ref, kvbuf, sk, rk, sv, rv, exit_sem):
        my_x = lax.axis_index("x")
        my_y = lax.axis_index("y")
        my_z = lax.axis_index("z")
        nbr = (1 - my_x, my_y, my_z)

        if comm:
            barrier_sem = pltpu.get_barrier_semaphore()
            pl.semaphore_signal(
                barrier_sem, inc=1, device_id=nbr,
                device_id_type=pl.DeviceIdType.MESH,
            )

        kvbuf[0, 0] = k_ref[...].astype(jnp.bfloat16)

        if comm:
            pl.semaphore_wait(barrier_sem, 1)
            rdma_k = pltpu.make_async_remote_copy(
                src_ref=kvbuf.at[0, 0], dst_ref=kvbuf.at[1, 0],
                send_sem=sk, recv_sem=rk,
                device_id=nbr, device_id_type=pl.DeviceIdType.MESH,
            )
            rdma_k.start()

        kvbuf[0, 1] = v_ref[...].astype(jnp.bfloat16)

        if comm:
            rdma_v = pltpu.make_async_remote_copy(
                src_ref=kvbuf.at[0, 1], dst_ref=kvbuf.at[1, 1],
                send_sem=sv, recv_sem=rv,
                device_id=nbr, device_id_type=pl.DeviceIdType.MESH,
            )
            rdma_v.start()

        if _KMODE == "nocompute":
            rdma_k.wait()
            rdma_v.wait()
            o_ref[0, 0] = kvbuf[1, 0, 0, 0].astype(jnp.float32)
            pl.semaphore_signal(
                exit_sem, inc=1, device_id=nbr,
                device_id_type=pl.DeviceIdType.MESH,
            )
            pl.semaphore_wait(exit_sem, 1)
            return

        dn_tn = (((0,), (0,)), ((), ()))
        dn_nn = (((1,), (0,)), ((), ()))

        def half_attn(slot, bi, hi, qt):
            kt = kvbuf[slot, 0, bi, hi]
            vt = kvbuf[slot, 1, bi, hi]
            st = lax.dot_general(kt, qt, dn_tn,
                                 preferred_element_type=jnp.float32)
            pt = jnp.exp(st)
            l = jnp.sum(pt, axis=0, keepdims=True)
            acc = lax.dot_general(vt, pt.astype(jnp.bfloat16), dn_nn,
                                  preferred_element_type=jnp.float32)
            return acc, l

        qts = []
        partial = []
        for bi in range(b):
            for hi in range(h):
                qt = (q_ref[bi, hi] * scale).astype(jnp.bfloat16)
                qts.append(qt)
                partial.append(half_attn(0, bi, hi, qt))

        if comm:
            rdma_k.wait()
            rdma_v.wait()

        idx = 0
        for bi in range(b):
            for hi in range(h):
                acc0, l0 = partial[idx]
                acc1, l1 = half_attn(1 if comm else 0, bi, hi, qts[idx])
                o_ref[bi, hi] = (acc0 + acc1) * (1.0 / (l0 + l1))
                idx += 1

        if comm:
            pl.semaphore_signal(
                exit_sem, inc=1, device_id=nbr,
                device_id_type=pl.DeviceIdType.MESH,
            )
            pl.semaphore_wait(exit_sem, 1)

    out_t = pl.pallas_call(
        body,
        out_shape=jax.ShapeDtypeStruct((b, h, d, s), jnp.float32),
        in_specs=[pl.BlockSpec(memory_space=pltpu.VMEM)] * 3,
        out_specs=pl.BlockSpec(memory_space=pltpu.VMEM),
        scratch_shapes=[
            pltpu.VMEM((2, 2, b, h, d, s), jnp.bfloat16),
            pltpu.SemaphoreType.DMA,
            pltpu.SemaphoreType.DMA,
            pltpu.SemaphoreType.DMA,
            pltpu.SemaphoreType.DMA,
            pltpu.SemaphoreType.REGULAR,
        ],
        compiler_params=pltpu.CompilerParams(
            vmem_limit_bytes=100 * 1024 * 1024,
            **({} if not comm else {"collective_id": 0}),
        ),
    )(Qt, Kt, Vt)
    return jnp.transpose(out_t, (0, 3, 1, 2))


# device time: 27384 ns/iter; 2.2159x vs baseline; 1.2673x over previous
import os

import jax
import jax.numpy as jnp
from jax import lax
from jax.experimental import pallas as pl
from jax.experimental.pallas import tpu as pltpu

_KMODE = os.environ.get("KMODE", "full")

_NCHUNK = 8


def kernel(Q, K, V):
    b, s, h, d = Q.shape
    scale = d ** -0.5
    comm = _KMODE != "nocomm"
    hq = h // 4

    Qt = jnp.transpose(Q, (0, 2, 3, 1))
    Kt = jnp.transpose(K, (0, 2, 3, 1))
    Vt = jnp.transpose(V, (0, 2, 3, 1))

    def body(q_ref, k_ref, v_ref, o_ref, kvbuf, sx, rx, sf, rf, exit_sem):
        my_x = lax.axis_index("x")
        my_y = lax.axis_index("y")
        my_z = lax.axis_index("z")
        xnbr = (1 - my_x, my_y, my_z)
        ynbr = (my_x, 1 - my_y, my_z)

        if comm:
            barrier_sem = pltpu.get_barrier_semaphore()
            for dev in (xnbr, ynbr):
                pl.semaphore_signal(
                    barrier_sem, inc=1, device_id=dev,
                    device_id_type=pl.DeviceIdType.MESH,
                )

        for bi in range(b):
            kvbuf[0, bi, 0] = k_ref[bi].astype(jnp.bfloat16)
            kvbuf[0, bi, 1] = v_ref[bi].astype(jnp.bfloat16)

        def chunk_at(slot, piece, c):
            kvi, hc = c // 4, c % 4
            return kvbuf.at[slot, piece, kvi, hc * hq:(hc + 1) * hq]

        rdma_x = []
        rdma_f = []
        if comm:
            pl.semaphore_wait(barrier_sem, 2)
            for c in range(_NCHUNK):
                r = pltpu.make_async_remote_copy(
                    src_ref=chunk_at(0, my_y, c),
                    dst_ref=chunk_at(1, my_y, c),
                    send_sem=sx.at[c], recv_sem=rx.at[c],
                    device_id=xnbr, device_id_type=pl.DeviceIdType.MESH,
                )
                r.start()
                rdma_x.append(r)

        dn_tn = (((0,), (0,)), ((), ()))
        dn_nn = (((1,), (0,)), ((), ()))

        def half_attn(slot, bi, hi, qt):
            kt = kvbuf[slot, bi, 0, hi]
            vt = kvbuf[slot, bi, 1, hi]
            st = lax.dot_general(kt, qt, dn_tn,
                                 preferred_element_type=jnp.float32)
            pt = jnp.exp(st)
            l = jnp.sum(pt, axis=0, keepdims=True)
            acc = lax.dot_general(vt, pt.astype(jnp.bfloat16), dn_nn,
                                  preferred_element_type=jnp.float32)
            return acc, l

        units = [(bi, hi) for bi in range(b) for hi in range(h)]
        per_chunk = len(units) // _NCHUNK

        qts = {}
        partial = {}

        def do_local(unit_idx):
            bi, hi = units[unit_idx]
            qt = (q_ref[bi, hi] * scale).astype(jnp.bfloat16)
            qts[(bi, hi)] = qt
            partial[(bi, hi)] = half_attn(0, bi, hi, qt)

        ui = 0
        if comm:
            for c in range(_NCHUNK):
                for _ in range(per_chunk):
                    do_local(ui)
                    ui += 1
                rdma_x[c].wait_recv()
                r = pltpu.make_async_remote_copy(
                    src_ref=chunk_at(1, my_y, c),
                    dst_ref=chunk_at(1, my_y, c),
                    send_sem=sf.at[c], recv_sem=rf.at[c],
                    device_id=ynbr, device_id_type=pl.DeviceIdType.MESH,
                )
                r.start()
                rdma_f.append(r)
        while ui < len(units):
            do_local(ui)
            ui += 1

        if comm:
            for c in range(_NCHUNK):
                rdma_x[c].wait_send()
                rdma_f[c].wait()

        if _KMODE == "nocompute":
            o_ref[0, 0] = kvbuf[1, 0, 0, 0].astype(jnp.float32)
        else:
            for bi in range(b):
                for hi in range(h):
                    acc0, l0 = partial[(bi, hi)]
                    acc1, l1 = half_attn(1 if comm else 0, bi, hi,
                                         qts[(bi, hi)])
                    o_ref[bi, hi] = (acc0 + acc1) * (1.0 / (l0 + l1))

        if comm:
            for dev in (xnbr, ynbr):
                pl.semaphore_signal(
                    exit_sem, inc=1, device_id=dev,
                    device_id_type=pl.DeviceIdType.MESH,
                )
            pl.semaphore_wait(exit_sem, 2)

    out_t = pl.pallas_call(
        body,
        out_shape=jax.ShapeDtypeStruct((b, h, d, s), jnp.float32),
        in_specs=[pl.BlockSpec(memory_space=pltpu.VMEM)] * 3,
        out_specs=pl.BlockSpec(memory_space=pltpu.VMEM),
        scratch_shapes=[
            pltpu.VMEM((2, b, 2, h, d, s), jnp.bfloat16),
            pltpu.SemaphoreType.DMA((_NCHUNK,)),
            pltpu.SemaphoreType.DMA((_NCHUNK,)),
            pltpu.SemaphoreType.DMA((_NCHUNK,)),
            pltpu.SemaphoreType.DMA((_NCHUNK,)),
            pltpu.SemaphoreType.REGULAR,
        ],
        compiler_params=pltpu.CompilerParams(
            vmem_limit_bytes=100 * 1024 * 1024,
            **({} if not comm else {"collective_id": 0}),
        ),
    )(Qt, Kt, Vt)
    return jnp.transpose(out_t, (0, 3, 1, 2))
